# baseline (speedup 1.0000x reference)
import dataclasses
import functools

import jax
import jax.numpy as jnp
from jax import lax
from jax.experimental import pallas as pl
from jax.experimental.pallas import tpu as pltpu
from jax.experimental.pallas import tpu_sc as plsc

NUM_CORES = 2
NUM_SUBCORES = 16
NUM_WORKERS = NUM_CORES * NUM_SUBCORES
BATCH = 16384
DIM = 64
BPW = BATCH // NUM_WORKERS
L = 16


def _glove_body(xi_hbm, xj_hbm, emb_hbm, bias_hbm, out_hbm,
                xi_v, xj_v, ri_v, rj_v, bqi_v, bqj_v, bi_v, bj_v, out_v,
                sem0, sem1, sem2, sem3):
    wid = lax.axis_index("subcore") * NUM_CORES + lax.axis_index("core")
    base = wid * BPW

    pltpu.sync_copy(xi_hbm.at[pl.ds(base, BPW)], xi_v)
    pltpu.sync_copy(xj_hbm.at[pl.ds(base, BPW)], xj_v)

    c0 = pltpu.async_copy(emb_hbm.at[xi_v], ri_v, sem0)
    c1 = pltpu.async_copy(emb_hbm.at[xj_v], rj_v, sem1)

    @pl.loop(0, BPW // L)
    def _(c):
        o = c * L
        bqi_v[pl.ds(o, L)] = jax.lax.shift_right_logical(xi_v[pl.ds(o, L)], 4)
        bqj_v[pl.ds(o, L)] = jax.lax.shift_right_logical(xj_v[pl.ds(o, L)], 4)

    c2 = pltpu.async_copy(bias_hbm.at[bqi_v], bi_v, sem2)
    c3 = pltpu.async_copy(bias_hbm.at[bqj_v], bj_v, sem3)
    c0.wait()
    c1.wait()
    c2.wait()
    c3.wait()

    lanes = lax.iota(jnp.int32, L)

    @pl.loop(0, BPW // L)
    def _(c):
        rows0 = c * L
        row_ids = lanes + rows0
        cols_i = xi_v[pl.ds(rows0, L)] & 15
        cols_j = xj_v[pl.ds(rows0, L)] & 15
        acc = (plsc.load_gather(bi_v, [row_ids, cols_i])
               + plsc.load_gather(bj_v, [row_ids, cols_j]))
        for r in range(L):
            off = rows0 + r
            s = (ri_v[off, pl.ds(0 * L, L)] * rj_v[off, pl.ds(0 * L, L)]
                 + ri_v[off, pl.ds(1 * L, L)] * rj_v[off, pl.ds(1 * L, L)]
                 + ri_v[off, pl.ds(2 * L, L)] * rj_v[off, pl.ds(2 * L, L)]
                 + ri_v[off, pl.ds(3 * L, L)] * rj_v[off, pl.ds(3 * L, L)])
            acc = acc + jnp.where(lanes == r, jnp.sum(s), 0.0)
        out_v[pl.ds(rows0, L)] = acc

    pltpu.sync_copy(out_v, out_hbm.at[pl.ds(base, BPW)])


@jax.jit
def kernel(x, embeddings, biases):
    xi = x[0]
    xj = x[1]
    bias_mat = biases.reshape(biases.shape[0] // L, L)
    mesh = plsc.VectorSubcoreMesh(core_axis_name="core",
                                  subcore_axis_name="subcore")
    cp = pltpu.CompilerParams(needs_layout_passes=False,
                              use_tc_tiling_on_sc=False)
    run = functools.partial(
        pl.kernel,
        mesh=mesh,
        compiler_params=cp,
        out_type=jax.ShapeDtypeStruct((BATCH,), jnp.float32),
        scratch_types=[
            pltpu.VMEM((BPW,), jnp.int32),
            pltpu.VMEM((BPW,), jnp.int32),
            pltpu.VMEM((BPW, DIM), jnp.float32),
            pltpu.VMEM((BPW, DIM), jnp.float32),
            pltpu.VMEM((BPW,), jnp.int32),
            pltpu.VMEM((BPW,), jnp.int32),
            pltpu.VMEM((BPW, L), jnp.float32),
            pltpu.VMEM((BPW, L), jnp.float32),
            pltpu.VMEM((BPW,), jnp.float32),
            pltpu.SemaphoreType.DMA,
            pltpu.SemaphoreType.DMA,
            pltpu.SemaphoreType.DMA,
            pltpu.SemaphoreType.DMA,
        ],
    )(_glove_body)
    return run(xi, xj, embeddings, bias_mat)


# speedup vs baseline: 1.0925x; 1.0925x over previous
import dataclasses
import functools

import jax
import jax.numpy as jnp
from jax import lax
from jax.experimental import pallas as pl
from jax.experimental.pallas import tpu as pltpu
from jax.experimental.pallas import tpu_sc as plsc

NUM_CORES = 2
NUM_SUBCORES = 16
NUM_WORKERS = NUM_CORES * NUM_SUBCORES
BATCH = 16384
DIM = 64
BPW = BATCH // NUM_WORKERS
L = 16
NCHUNK = 8
CH = BPW // NCHUNK
NBUF = 4


def _glove_body(xi_hbm, xj_hbm, emb_hbm, bias_hbm, out_hbm,
                xi_v, xj_v, ri_bufs, rj_bufs, bqi_v, bqj_v, bi_v, bj_v, out_v,
                sems, bsem0, bsem1):
    wid = lax.axis_index("subcore") * NUM_CORES + lax.axis_index("core")
    base = wid * BPW

    pltpu.sync_copy(xi_hbm.at[pl.ds(base, BPW)], xi_v)
    pltpu.sync_copy(xj_hbm.at[pl.ds(base, BPW)], xj_v)

    def issue(c):
        o = c * CH
        b = c % NBUF
        return (
            pltpu.async_copy(emb_hbm.at[xi_v.at[pl.ds(o, CH)]],
                             ri_bufs.at[b], sems.at[2 * b]),
            pltpu.async_copy(emb_hbm.at[xj_v.at[pl.ds(o, CH)]],
                             rj_bufs.at[b], sems.at[2 * b + 1]),
        )

    copies = {c: issue(c) for c in range(NBUF)}

    @pl.loop(0, BPW // L)
    def _(c):
        o = c * L
        bqi_v[pl.ds(o, L)] = jax.lax.shift_right_logical(xi_v[pl.ds(o, L)], 4)
        bqj_v[pl.ds(o, L)] = jax.lax.shift_right_logical(xj_v[pl.ds(o, L)], 4)

    cbi = pltpu.async_copy(bias_hbm.at[bqi_v], bi_v, bsem0)
    cbj = pltpu.async_copy(bias_hbm.at[bqj_v], bj_v, bsem1)
    cbi.wait()
    cbj.wait()

    lanes = lax.iota(jnp.int32, L)

    @pl.loop(0, BPW // L)
    def _(c):
        rows0 = c * L
        row_ids = lanes + rows0
        cols_i = xi_v[pl.ds(rows0, L)] & 15
        cols_j = xj_v[pl.ds(rows0, L)] & 15
        out_v[pl.ds(rows0, L)] = (plsc.load_gather(bi_v, [row_ids, cols_i])
                                  + plsc.load_gather(bj_v, [row_ids, cols_j]))

    for c in range(NCHUNK):
        copies[c][0].wait()
        copies[c][1].wait()
        ri_v = ri_bufs.at[c % NBUF]
        rj_v = rj_bufs.at[c % NBUF]

        @pl.loop(0, CH // L)
        def _(blk):
            rows0 = blk * L
            for r in range(L):
                loc = rows0 + r
                s = (ri_v[loc, pl.ds(0 * L, L)] * rj_v[loc, pl.ds(0 * L, L)]
                     + ri_v[loc, pl.ds(1 * L, L)] * rj_v[loc, pl.ds(1 * L, L)]
                     + ri_v[loc, pl.ds(2 * L, L)] * rj_v[loc, pl.ds(2 * L, L)]
                     + ri_v[loc, pl.ds(3 * L, L)] * rj_v[loc, pl.ds(3 * L, L)])
                row_vec = jnp.full((L,), c * CH + loc, jnp.int32)
                plsc.addupdate_scatter(out_v, [row_vec], s)

        if c + NBUF < NCHUNK:
            copies[c + NBUF] = issue(c + NBUF)

    pltpu.sync_copy(out_v, out_hbm.at[pl.ds(base, BPW)])


@jax.jit
def kernel(x, embeddings, biases):
    xi = x[0]
    xj = x[1]
    emb128 = jnp.pad(embeddings, ((0, 0), (0, 128 - DIM)))
    bias_mat = biases.reshape(biases.shape[0] // L, L)
    mesh = plsc.VectorSubcoreMesh(core_axis_name="core",
                                  subcore_axis_name="subcore")
    cp = pltpu.CompilerParams(needs_layout_passes=False,
                              use_tc_tiling_on_sc=False)
    run = functools.partial(
        pl.kernel,
        mesh=mesh,
        compiler_params=cp,
        out_type=jax.ShapeDtypeStruct((BATCH,), jnp.float32),
        scratch_types=[
            pltpu.VMEM((BPW,), jnp.int32),
            pltpu.VMEM((BPW,), jnp.int32),
            pltpu.VMEM((NBUF, CH, 128), jnp.float32),
            pltpu.VMEM((NBUF, CH, 128), jnp.float32),
            pltpu.VMEM((BPW,), jnp.int32),
            pltpu.VMEM((BPW,), jnp.int32),
            pltpu.VMEM((BPW, L), jnp.float32),
            pltpu.VMEM((BPW, L), jnp.float32),
            pltpu.VMEM((BPW,), jnp.float32),
            pltpu.SemaphoreType.DMA((2 * NCHUNK,)),
            pltpu.SemaphoreType.DMA,
            pltpu.SemaphoreType.DMA,
        ],
    )(_glove_body)
    return run(xi, xj, emb128, bias_mat)


# speedup vs baseline: 1.4829x; 1.3573x over previous
import functools

import jax
import jax.numpy as jnp
from jax import lax
from jax.experimental import pallas as pl
from jax.experimental.pallas import tpu as pltpu
from jax.experimental.pallas import tpu_sc as plsc

NUM_CORES = 2
NUM_SUBCORES = 16
NUM_WORKERS = NUM_CORES * NUM_SUBCORES
BATCH = 16384
DIM = 64
BPW = BATCH // NUM_WORKERS
L = 16
NCHUNK = 8
CH = BPW // NCHUNK
NBUF = 4


def _glove_body(xi_hbm, xj_hbm, emb_hbm, bias_hbm, out_hbm,
                xi_v, xj_v, ri_bufs, rj_bufs, bi_v, bj_v, out_v,
                sems, bsem):
    wid = lax.axis_index("subcore") * NUM_CORES + lax.axis_index("core")
    base = wid * BPW

    pltpu.sync_copy(xi_hbm.at[pl.ds(base, BPW)], xi_v)
    pltpu.sync_copy(xj_hbm.at[pl.ds(base, BPW)], xj_v)

    def issue(c):
        o = c * CH
        b = c % NBUF

        @pl.loop(0, CH // L)
        def _(g):
            gg = g * L
            vi = xi_v[pl.ds(o + gg, L)]
            vj = xj_v[pl.ds(o + gg, L)]
            for r in range(L):
                loc = gg + r
                pltpu.async_copy(emb_hbm.at[pl.ds(vi[r], 1)],
                                 ri_bufs.at[b, pl.ds(loc, 1)], sems.at[2 * b])
                pltpu.async_copy(emb_hbm.at[pl.ds(vj[r], 1)],
                                 rj_bufs.at[b, pl.ds(loc, 1)],
                                 sems.at[2 * b + 1])
                dst_off = pl.multiple_of((o + loc) * L, L)
                pltpu.async_copy(
                    bias_hbm.at[pl.ds(pl.multiple_of(vi[r] & -16, L), L)],
                    bi_v.at[pl.ds(dst_off, L)], bsem)
                pltpu.async_copy(
                    bias_hbm.at[pl.ds(pl.multiple_of(vj[r] & -16, L), L)],
                    bj_v.at[pl.ds(dst_off, L)], bsem)

    def drain(c):
        b = c % NBUF
        pltpu.make_async_copy(emb_hbm.at[pl.ds(0, CH)], ri_bufs.at[b],
                              sems.at[2 * b]).wait()
        pltpu.make_async_copy(emb_hbm.at[pl.ds(0, CH)], rj_bufs.at[b],
                              sems.at[2 * b + 1]).wait()

    for c in range(NBUF):
        issue(c)

    zeros16 = jnp.zeros((L,), jnp.float32)

    @pl.loop(0, BPW // L)
    def _(blk):
        out_v[pl.ds(blk * L, L)] = zeros16

    for c in range(NCHUNK):
        drain(c)
        ri_v = ri_bufs.at[c % NBUF]
        rj_v = rj_bufs.at[c % NBUF]

        @pl.loop(0, CH // L)
        def _(blk):
            rows0 = blk * L
            for r in range(L):
                loc = rows0 + r
                s = (ri_v[loc, pl.ds(0 * L, L)] * rj_v[loc, pl.ds(0 * L, L)]
                     + ri_v[loc, pl.ds(1 * L, L)] * rj_v[loc, pl.ds(1 * L, L)]
                     + ri_v[loc, pl.ds(2 * L, L)] * rj_v[loc, pl.ds(2 * L, L)]
                     + ri_v[loc, pl.ds(3 * L, L)] * rj_v[loc, pl.ds(3 * L, L)])
                row_vec = jnp.full((L,), c * CH + loc, jnp.int32)
                plsc.addupdate_scatter(out_v, [row_vec], s)

        if c + NBUF < NCHUNK:
            issue(c + NBUF)

    pltpu.make_async_copy(bias_hbm.at[pl.ds(0, BPW * L)], bi_v, bsem).wait()
    pltpu.make_async_copy(bias_hbm.at[pl.ds(0, BPW * L)], bj_v, bsem).wait()

    lanes = lax.iota(jnp.int32, L)

    @pl.loop(0, BPW // L)
    def _(blk):
        o = blk * L
        pos = (lanes + o) * L
        cols_i = pos + (xi_v[pl.ds(o, L)] & 15)
        cols_j = pos + (xj_v[pl.ds(o, L)] & 15)
        out_v[pl.ds(o, L)] = (out_v[pl.ds(o, L)]
                              + plsc.load_gather(bi_v, [cols_i])
                              + plsc.load_gather(bj_v, [cols_j]))

    pltpu.sync_copy(out_v, out_hbm.at[pl.ds(base, BPW)])


@jax.jit
def kernel(x, embeddings, biases):
    xi = x[0]
    xj = x[1]
    bias_flat = biases.reshape(-1)
    mesh = plsc.VectorSubcoreMesh(core_axis_name="core",
                                  subcore_axis_name="subcore")
    cp = pltpu.CompilerParams(needs_layout_passes=False)
    run = functools.partial(
        pl.kernel,
        mesh=mesh,
        compiler_params=cp,
        out_type=jax.ShapeDtypeStruct((BATCH,), jnp.float32),
        scratch_types=[
            pltpu.VMEM((BPW,), jnp.int32),
            pltpu.VMEM((BPW,), jnp.int32),
            pltpu.VMEM((NBUF, CH, DIM), jnp.float32),
            pltpu.VMEM((NBUF, CH, DIM), jnp.float32),
            pltpu.VMEM((BPW * L,), jnp.float32),
            pltpu.VMEM((BPW * L,), jnp.float32),
            pltpu.VMEM((BPW,), jnp.float32),
            pltpu.SemaphoreType.DMA((2 * NBUF,)),
            pltpu.SemaphoreType.DMA,
        ],
    )(_glove_body)
    return run(xi, xj, embeddings, bias_flat)
